# baseline (speedup 1.0000x reference)
import functools

import jax
import jax.numpy as jnp
from jax import lax
from jax.experimental import pallas as pl
from jax.experimental.pallas import tpu as pltpu
from jax.experimental.pallas import tpu_sc as plsc

N = 10000
E = 320000
D = 128

NC = 2
NS = 16
CHUNK = 72
NCHUNK = 140
EDGES_PER_TILE = NCHUNK * CHUNK
GC = 28
NGROUPS = NCHUNK // GC
NP = 10240
ROWS_PER_TILE = NP // NS


def _softmax_body(x_ref, o_ref):
    x = x_ref[...]
    m = jnp.max(x, axis=-1, keepdims=True)
    e = jnp.exp(x - m)
    o_ref[...] = e / jnp.sum(e, axis=-1, keepdims=True)


def _softmax(x):
    br = 2000
    return pl.pallas_call(
        _softmax_body,
        grid=(N // br,),
        in_specs=[pl.BlockSpec((br, D), lambda i: (i, 0))],
        out_specs=pl.BlockSpec((br, D), lambda i: (i, 0)),
        out_shape=jax.ShapeDtypeStruct((N, D), jnp.float32),
    )(x)


def _sc_segment_sum(p, src_r, dst_r, zeros):
    mesh = plsc.VectorSubcoreMesh(core_axis_name="c", subcore_axis_name="s")

    @functools.partial(
        pl.kernel,
        out_type=jax.ShapeDtypeStruct((NC, NP, D), jnp.float32),
        mesh=mesh,
        scratch_types=[
            pltpu.VMEM((GC, CHUNK), jnp.int32),
            pltpu.VMEM((GC, CHUNK), jnp.int32),
            pltpu.VMEM((CHUNK, D), jnp.float32),
            pltpu.VMEM_SHARED((NP, D), jnp.float32),
        ],
    )
    def k(p_hbm, src_hbm, dst_hbm, z_hbm, out_hbm, src_g, dst_g, rows_v,
          acc_sh):
        c = lax.axis_index("c")
        s = lax.axis_index("s")
        r0 = s * ROWS_PER_TILE
        pltpu.sync_copy(z_hbm, acc_sh.at[pl.ds(r0, ROWS_PER_TILE)])
        plsc.subcore_barrier()

        for g in range(NGROUPS):
            pltpu.sync_copy(src_hbm.at[c, s, g], src_g)
            pltpu.sync_copy(dst_hbm.at[c, s, g], dst_g)

            @pl.loop(0, GC)
            def _(i):
                pltpu.sync_copy(p_hbm.at[src_g.at[i]], rows_v)
                pltpu.sync_copy(rows_v, acc_sh.at[dst_g.at[i]], add=True)

        plsc.subcore_barrier()
        pltpu.sync_copy(acc_sh.at[pl.ds(r0, ROWS_PER_TILE)],
                        out_hbm.at[c, pl.ds(r0, ROWS_PER_TILE)])

    return k(p, src_r, dst_r, zeros)


def _mm_body(g_ref, wp_ref, wn_ref, o_ref):
    w_eff = wp_ref[...] - jax.nn.softplus(wn_ref[...])
    g = g_ref[0] + g_ref[1]
    o_ref[...] = jnp.dot(g, w_eff, preferred_element_type=jnp.float32)


def _combine_matmul(g, w_pos, w_neg_raw):
    br = 2048
    return pl.pallas_call(
        _mm_body,
        grid=(NP // br,),
        in_specs=[
            pl.BlockSpec((NC, br, D), lambda i: (0, i, 0)),
            pl.BlockSpec((D, D), lambda i: (0, 0)),
            pl.BlockSpec((D, D), lambda i: (0, 0)),
        ],
        out_specs=pl.BlockSpec((br, D), lambda i: (i, 0)),
        out_shape=jax.ShapeDtypeStruct((NP, D), jnp.float32),
    )(g, w_pos, w_neg_raw)


@jax.jit
def kernel(x_src, edge_index, frozen_src, W_pos, W_neg_raw):
    del frozen_src
    p = _softmax(x_src)
    nw = NC * NS
    real = E // nw
    pad = EDGES_PER_TILE - real
    src_r = jnp.concatenate(
        [edge_index[0].reshape(nw, real),
         jnp.zeros((nw, pad), jnp.int32)], axis=1,
    ).reshape(NC, NS, NGROUPS, GC, CHUNK)
    sink = N + jnp.arange(nw, dtype=jnp.int32)[:, None]
    dst_r = jnp.concatenate(
        [edge_index[1].reshape(nw, real),
         jnp.broadcast_to(sink, (nw, pad)).astype(jnp.int32)], axis=1,
    ).reshape(NC, NS, NGROUPS, GC, CHUNK)
    zeros = jnp.zeros((ROWS_PER_TILE, D), dtype=jnp.float32)
    g = _sc_segment_sum(p, src_r, dst_r, zeros)
    return _combine_matmul(g, W_pos, W_neg_raw)[:N]


# speedup vs baseline: 1.4810x; 1.4810x over previous
import functools

import jax
import jax.numpy as jnp
from jax import lax
from jax.experimental import pallas as pl
from jax.experimental.pallas import tpu as pltpu
from jax.experimental.pallas import tpu_sc as plsc

N = 10000
E = 320000
D = 128

NC = 2
NS = 16
CHUNK = 100
NCHUNK = 100
EDGES_PER_TILE = NCHUNK * CHUNK
GC = 25
NGROUPS = NCHUNK // GC
NP = 10240
ROWS_PER_TILE = NP // NS


def _softmax_body(x_ref, o_ref):
    x = x_ref[...]
    m = jnp.max(x, axis=-1, keepdims=True)
    e = jnp.exp(x - m)
    o_ref[...] = e / jnp.sum(e, axis=-1, keepdims=True)


def _softmax(x):
    br = 2000
    return pl.pallas_call(
        _softmax_body,
        grid=(N // br,),
        in_specs=[pl.BlockSpec((br, D), lambda i: (i, 0))],
        out_specs=pl.BlockSpec((br, D), lambda i: (i, 0)),
        out_shape=jax.ShapeDtypeStruct((N, D), jnp.float32),
    )(x)


def _sc_segment_sum(p, src_r, dst_r, zeros):
    mesh = plsc.VectorSubcoreMesh(core_axis_name="c", subcore_axis_name="s")

    @functools.partial(
        pl.kernel,
        out_type=jax.ShapeDtypeStruct((NC, NP, D), jnp.float32),
        mesh=mesh,
        scratch_types=[
            pltpu.VMEM((GC, CHUNK), jnp.int32),
            pltpu.VMEM((GC, CHUNK), jnp.int32),
            pltpu.VMEM((CHUNK, D), jnp.float32),
            pltpu.VMEM_SHARED((NP, D), jnp.float32),
        ],
    )
    def k(p_hbm, src_hbm, dst_hbm, z_hbm, out_hbm, src_g, dst_g, rows_v,
          acc_sh):
        c = lax.axis_index("c")
        s = lax.axis_index("s")
        r0 = s * ROWS_PER_TILE
        pltpu.sync_copy(z_hbm, acc_sh.at[pl.ds(r0, ROWS_PER_TILE)])
        plsc.subcore_barrier()

        for g in range(NGROUPS):
            pltpu.sync_copy(src_hbm.at[c, s, g], src_g)
            pltpu.sync_copy(dst_hbm.at[c, s, g], dst_g)

            @pl.loop(0, GC)
            def _(i):
                pltpu.sync_copy(p_hbm.at[src_g.at[i]], rows_v)
                pltpu.sync_copy(rows_v, acc_sh.at[dst_g.at[i]], add=True)

        plsc.subcore_barrier()
        pltpu.sync_copy(acc_sh.at[pl.ds(r0, ROWS_PER_TILE)],
                        out_hbm.at[c, pl.ds(r0, ROWS_PER_TILE)])

    return k(p, src_r, dst_r, zeros)


def _mm_body(g_ref, wp_ref, wn_ref, o_ref):
    w_eff = wp_ref[...] - jax.nn.softplus(wn_ref[...])
    g = g_ref[0] + g_ref[1]
    o_ref[...] = jnp.dot(g, w_eff, preferred_element_type=jnp.float32)


def _combine_matmul(g, w_pos, w_neg_raw):
    br = 2048
    return pl.pallas_call(
        _mm_body,
        grid=(NP // br,),
        in_specs=[
            pl.BlockSpec((NC, br, D), lambda i: (0, i, 0)),
            pl.BlockSpec((D, D), lambda i: (0, 0)),
            pl.BlockSpec((D, D), lambda i: (0, 0)),
        ],
        out_specs=pl.BlockSpec((br, D), lambda i: (i, 0)),
        out_shape=jax.ShapeDtypeStruct((NP, D), jnp.float32),
    )(g, w_pos, w_neg_raw)


@jax.jit
def kernel(x_src, edge_index, frozen_src, W_pos, W_neg_raw):
    del frozen_src
    p = _softmax(x_src)
    nw = NC * NS
    real = E // nw
    pad = EDGES_PER_TILE - real
    src_r = jnp.concatenate(
        [edge_index[0].reshape(nw, real),
         jnp.zeros((nw, pad), jnp.int32)], axis=1,
    ).reshape(NC, NS, NGROUPS, GC, CHUNK)
    sink = N + jnp.arange(nw, dtype=jnp.int32)[:, None]
    dst_r = jnp.concatenate(
        [edge_index[1].reshape(nw, real),
         jnp.broadcast_to(sink, (nw, pad)).astype(jnp.int32)], axis=1,
    ).reshape(NC, NS, NGROUPS, GC, CHUNK)
    zeros = jnp.zeros((ROWS_PER_TILE, D), dtype=jnp.float32)
    g = _sc_segment_sum(p, src_r, dst_r, zeros)
    return _combine_matmul(g, W_pos, W_neg_raw)[:N]


# speedup vs baseline: 1.5728x; 1.0620x over previous
import functools

import jax
import jax.numpy as jnp
from jax import lax
from jax.experimental import pallas as pl
from jax.experimental.pallas import tpu as pltpu
from jax.experimental.pallas import tpu_sc as plsc

N = 10000
E = 320000
D = 128

NC = 2
NS = 16
CHUNK = 125
NCHUNK = 80
EDGES_PER_TILE = NCHUNK * CHUNK
GC = 20
NGROUPS = NCHUNK // GC
NP = 10240
ROWS_PER_TILE = NP // NS


def _softmax_body(x_ref, o_ref):
    x = x_ref[...]
    m = jnp.max(x, axis=-1, keepdims=True)
    e = jnp.exp(x - m)
    o_ref[...] = e / jnp.sum(e, axis=-1, keepdims=True)


def _softmax(x):
    br = 2000
    return pl.pallas_call(
        _softmax_body,
        grid=(N // br,),
        in_specs=[pl.BlockSpec((br, D), lambda i: (i, 0))],
        out_specs=pl.BlockSpec((br, D), lambda i: (i, 0)),
        out_shape=jax.ShapeDtypeStruct((N, D), jnp.float32),
    )(x)


def _sc_segment_sum(p, src_r, dst_r, zeros):
    mesh = plsc.VectorSubcoreMesh(core_axis_name="c", subcore_axis_name="s")

    @functools.partial(
        pl.kernel,
        out_type=jax.ShapeDtypeStruct((NC, NP, D), jnp.float32),
        mesh=mesh,
        scratch_types=[
            pltpu.VMEM((GC, CHUNK), jnp.int32),
            pltpu.VMEM((GC, CHUNK), jnp.int32),
            pltpu.VMEM((CHUNK, D), jnp.float32),
            pltpu.VMEM_SHARED((NP, D), jnp.float32),
        ],
    )
    def k(p_hbm, src_hbm, dst_hbm, z_hbm, out_hbm, src_g, dst_g, rows_v,
          acc_sh):
        c = lax.axis_index("c")
        s = lax.axis_index("s")
        r0 = s * ROWS_PER_TILE
        pltpu.sync_copy(z_hbm, acc_sh.at[pl.ds(r0, ROWS_PER_TILE)])
        plsc.subcore_barrier()

        for g in range(NGROUPS):
            pltpu.sync_copy(src_hbm.at[c, s, g], src_g)
            pltpu.sync_copy(dst_hbm.at[c, s, g], dst_g)

            @pl.loop(0, GC)
            def _(i):
                pltpu.sync_copy(p_hbm.at[src_g.at[i]], rows_v)
                pltpu.sync_copy(rows_v, acc_sh.at[dst_g.at[i]], add=True)

        plsc.subcore_barrier()
        pltpu.sync_copy(acc_sh.at[pl.ds(r0, ROWS_PER_TILE)],
                        out_hbm.at[c, pl.ds(r0, ROWS_PER_TILE)])

    return k(p, src_r, dst_r, zeros)


def _mm_body(g_ref, wp_ref, wn_ref, o_ref):
    w_eff = wp_ref[...] - jax.nn.softplus(wn_ref[...])
    g = g_ref[0] + g_ref[1]
    o_ref[...] = jnp.dot(g, w_eff, preferred_element_type=jnp.float32)


def _combine_matmul(g, w_pos, w_neg_raw):
    br = 2048
    return pl.pallas_call(
        _mm_body,
        grid=(NP // br,),
        in_specs=[
            pl.BlockSpec((NC, br, D), lambda i: (0, i, 0)),
            pl.BlockSpec((D, D), lambda i: (0, 0)),
            pl.BlockSpec((D, D), lambda i: (0, 0)),
        ],
        out_specs=pl.BlockSpec((br, D), lambda i: (i, 0)),
        out_shape=jax.ShapeDtypeStruct((NP, D), jnp.float32),
    )(g, w_pos, w_neg_raw)


@jax.jit
def kernel(x_src, edge_index, frozen_src, W_pos, W_neg_raw):
    del frozen_src
    p = _softmax(x_src)
    nw = NC * NS
    real = E // nw
    pad = EDGES_PER_TILE - real
    src_r = jnp.concatenate(
        [edge_index[0].reshape(nw, real),
         jnp.zeros((nw, pad), jnp.int32)], axis=1,
    ).reshape(NC, NS, NGROUPS, GC, CHUNK)
    sink = N + jnp.arange(nw, dtype=jnp.int32)[:, None]
    dst_r = jnp.concatenate(
        [edge_index[1].reshape(nw, real),
         jnp.broadcast_to(sink, (nw, pad)).astype(jnp.int32)], axis=1,
    ).reshape(NC, NS, NGROUPS, GC, CHUNK)
    zeros = jnp.zeros((ROWS_PER_TILE, D), dtype=jnp.float32)
    g = _sc_segment_sum(p, src_r, dst_r, zeros)
    return _combine_matmul(g, W_pos, W_neg_raw)[:N]


# speedup vs baseline: 1.9458x; 1.2371x over previous
import functools

import jax
import jax.numpy as jnp
from jax import lax
from jax.experimental import pallas as pl
from jax.experimental.pallas import tpu as pltpu
from jax.experimental.pallas import tpu_sc as plsc

N = 10000
E = 320000
D = 128

NC = 2
NS = 16
CHUNK = 125
NCHUNK = 80
EDGES_PER_TILE = NCHUNK * CHUNK
GC = 20
NGROUPS = NCHUNK // GC
NP = 10240
ROWS_PER_TILE = NP // NS


def _softmax_body(x_ref, o_ref):
    x = x_ref[...]
    m = jnp.max(x, axis=-1, keepdims=True)
    e = jnp.exp(x - m)
    o_ref[...] = e / jnp.sum(e, axis=-1, keepdims=True)


def _softmax(x):
    br = 2000
    return pl.pallas_call(
        _softmax_body,
        grid=(N // br,),
        in_specs=[pl.BlockSpec((br, D), lambda i: (i, 0))],
        out_specs=pl.BlockSpec((br, D), lambda i: (i, 0)),
        out_shape=jax.ShapeDtypeStruct((N, D), jnp.float32),
    )(x)


def _sc_segment_sum(p, src_r, dst_r, zeros):
    mesh = plsc.VectorSubcoreMesh(core_axis_name="c", subcore_axis_name="s")

    @functools.partial(
        pl.kernel,
        out_type=jax.ShapeDtypeStruct((NC, NP, D), jnp.float32),
        mesh=mesh,
        scratch_types=[
            pltpu.VMEM((GC, CHUNK), jnp.int32),
            pltpu.VMEM((GC, CHUNK), jnp.int32),
            pltpu.VMEM((2, CHUNK, D), jnp.float32),
            pltpu.VMEM_SHARED((NP, D), jnp.float32),
            pltpu.SemaphoreType.DMA,
        ],
    )
    def k(p_hbm, src_hbm, dst_hbm, z_hbm, out_hbm, src_g, dst_g, rows_v,
          acc_sh, sem):
        c = lax.axis_index("c")
        s = lax.axis_index("s")
        r0 = s * ROWS_PER_TILE
        pltpu.sync_copy(z_hbm, acc_sh.at[pl.ds(r0, ROWS_PER_TILE)])
        plsc.subcore_barrier()

        for g in range(NGROUPS):
            pltpu.sync_copy(src_hbm.at[c, s, g], src_g)
            pltpu.sync_copy(dst_hbm.at[c, s, g], dst_g)

            pltpu.async_copy(p_hbm.at[src_g.at[0]], rows_v.at[0], sem)

            @pl.loop(0, GC)
            def _(i):
                b = lax.rem(i, 2)
                pltpu.make_async_copy(p_hbm.at[src_g.at[i]], rows_v.at[b],
                                      sem).wait()

                @pl.when(i + 1 < GC)
                def _():
                    pltpu.async_copy(p_hbm.at[src_g.at[i + 1]],
                                     rows_v.at[1 - b], sem)

                pltpu.sync_copy(rows_v.at[b], acc_sh.at[dst_g.at[i]],
                                add=True)

        plsc.subcore_barrier()
        pltpu.sync_copy(acc_sh.at[pl.ds(r0, ROWS_PER_TILE)],
                        out_hbm.at[c, pl.ds(r0, ROWS_PER_TILE)])

    return k(p, src_r, dst_r, zeros)


def _mm_body(g_ref, wp_ref, wn_ref, o_ref):
    w_eff = wp_ref[...] - jax.nn.softplus(wn_ref[...])
    g = g_ref[0] + g_ref[1]
    o_ref[...] = jnp.dot(g, w_eff, preferred_element_type=jnp.float32)


def _combine_matmul(g, w_pos, w_neg_raw):
    br = 2048
    return pl.pallas_call(
        _mm_body,
        grid=(NP // br,),
        in_specs=[
            pl.BlockSpec((NC, br, D), lambda i: (0, i, 0)),
            pl.BlockSpec((D, D), lambda i: (0, 0)),
            pl.BlockSpec((D, D), lambda i: (0, 0)),
        ],
        out_specs=pl.BlockSpec((br, D), lambda i: (i, 0)),
        out_shape=jax.ShapeDtypeStruct((NP, D), jnp.float32),
    )(g, w_pos, w_neg_raw)


@jax.jit
def kernel(x_src, edge_index, frozen_src, W_pos, W_neg_raw):
    del frozen_src
    p = _softmax(x_src)
    nw = NC * NS
    real = E // nw
    pad = EDGES_PER_TILE - real
    src_r = jnp.concatenate(
        [edge_index[0].reshape(nw, real),
         jnp.zeros((nw, pad), jnp.int32)], axis=1,
    ).reshape(NC, NS, NGROUPS, GC, CHUNK)
    sink = N + jnp.arange(nw, dtype=jnp.int32)[:, None]
    dst_r = jnp.concatenate(
        [edge_index[1].reshape(nw, real),
         jnp.broadcast_to(sink, (nw, pad)).astype(jnp.int32)], axis=1,
    ).reshape(NC, NS, NGROUPS, GC, CHUNK)
    zeros = jnp.zeros((ROWS_PER_TILE, D), dtype=jnp.float32)
    g = _sc_segment_sum(p, src_r, dst_r, zeros)
    return _combine_matmul(g, W_pos, W_neg_raw)[:N]


# speedup vs baseline: 2.3068x; 1.1856x over previous
import functools

import jax
import jax.numpy as jnp
from jax import lax
from jax.experimental import pallas as pl
from jax.experimental.pallas import tpu as pltpu
from jax.experimental.pallas import tpu_sc as plsc

N = 10000
E = 320000
D = 128

NC = 2
NS = 16
CHUNK = 100
NCHUNK = 100
EDGES_PER_TILE = NCHUNK * CHUNK
GC = 20
NGROUPS = NCHUNK // GC
NP = 10240
ROWS_PER_TILE = NP // NS


def _softmax_body(x_ref, o_ref):
    x = x_ref[...]
    m = jnp.max(x, axis=-1, keepdims=True)
    e = jnp.exp(x - m)
    o_ref[...] = e / jnp.sum(e, axis=-1, keepdims=True)


def _softmax(x):
    br = 2000
    return pl.pallas_call(
        _softmax_body,
        grid=(N // br,),
        in_specs=[pl.BlockSpec((br, D), lambda i: (i, 0))],
        out_specs=pl.BlockSpec((br, D), lambda i: (i, 0)),
        out_shape=jax.ShapeDtypeStruct((N, D), jnp.float32),
    )(x)


def _sc_segment_sum(p, src_r, dst_r, zeros):
    mesh = plsc.VectorSubcoreMesh(core_axis_name="c", subcore_axis_name="s")

    @functools.partial(
        pl.kernel,
        out_type=jax.ShapeDtypeStruct((NC, NP, D), jnp.float32),
        mesh=mesh,
        scratch_types=[
            pltpu.VMEM((GC, CHUNK), jnp.int32),
            pltpu.VMEM((GC, CHUNK), jnp.int32),
            pltpu.VMEM((3, CHUNK, D), jnp.float32),
            pltpu.VMEM_SHARED((NP, D), jnp.float32),
            pltpu.SemaphoreType.DMA,
        ],
    )
    def k(p_hbm, src_hbm, dst_hbm, z_hbm, out_hbm, src_g, dst_g, rows_v,
          acc_sh, sem):
        c = lax.axis_index("c")
        s = lax.axis_index("s")
        r0 = s * ROWS_PER_TILE
        pltpu.sync_copy(z_hbm, acc_sh.at[pl.ds(r0, ROWS_PER_TILE)])
        plsc.subcore_barrier()

        for g in range(NGROUPS):
            pltpu.sync_copy(src_hbm.at[c, s, g], src_g)
            pltpu.sync_copy(dst_hbm.at[c, s, g], dst_g)

            pltpu.async_copy(p_hbm.at[src_g.at[0]], rows_v.at[0], sem)
            pltpu.async_copy(p_hbm.at[src_g.at[1]], rows_v.at[1], sem)

            @pl.loop(0, GC)
            def _(i):
                b = lax.rem(i, 3)
                pltpu.make_async_copy(p_hbm.at[src_g.at[i]], rows_v.at[b],
                                      sem).wait()

                @pl.when(i + 2 < GC)
                def _():
                    pltpu.async_copy(p_hbm.at[src_g.at[i + 2]],
                                     rows_v.at[lax.rem(i + 2, 3)], sem)

                pltpu.sync_copy(rows_v.at[b], acc_sh.at[dst_g.at[i]],
                                add=True)

        plsc.subcore_barrier()
        pltpu.sync_copy(acc_sh.at[pl.ds(r0, ROWS_PER_TILE)],
                        out_hbm.at[c, pl.ds(r0, ROWS_PER_TILE)])

    return k(p, src_r, dst_r, zeros)


def _mm_body(g_ref, wp_ref, wn_ref, o_ref):
    w_eff = wp_ref[...] - jax.nn.softplus(wn_ref[...])
    g = g_ref[0] + g_ref[1]
    o_ref[...] = jnp.dot(g, w_eff, preferred_element_type=jnp.float32)


def _combine_matmul(g, w_pos, w_neg_raw):
    br = 2048
    return pl.pallas_call(
        _mm_body,
        grid=(NP // br,),
        in_specs=[
            pl.BlockSpec((NC, br, D), lambda i: (0, i, 0)),
            pl.BlockSpec((D, D), lambda i: (0, 0)),
            pl.BlockSpec((D, D), lambda i: (0, 0)),
        ],
        out_specs=pl.BlockSpec((br, D), lambda i: (i, 0)),
        out_shape=jax.ShapeDtypeStruct((NP, D), jnp.float32),
    )(g, w_pos, w_neg_raw)


@jax.jit
def kernel(x_src, edge_index, frozen_src, W_pos, W_neg_raw):
    del frozen_src
    p = _softmax(x_src)
    nw = NC * NS
    real = E // nw
    pad = EDGES_PER_TILE - real
    src_r = jnp.concatenate(
        [edge_index[0].reshape(nw, real),
         jnp.zeros((nw, pad), jnp.int32)], axis=1,
    ).reshape(NC, NS, NGROUPS, GC, CHUNK)
    sink = N + jnp.arange(nw, dtype=jnp.int32)[:, None]
    dst_r = jnp.concatenate(
        [edge_index[1].reshape(nw, real),
         jnp.broadcast_to(sink, (nw, pad)).astype(jnp.int32)], axis=1,
    ).reshape(NC, NS, NGROUPS, GC, CHUNK)
    zeros = jnp.zeros((ROWS_PER_TILE, D), dtype=jnp.float32)
    g = _sc_segment_sum(p, src_r, dst_r, zeros)
    return _combine_matmul(g, W_pos, W_neg_raw)[:N]
